# baseline (speedup 1.0000x reference)
import functools

import jax
import jax.numpy as jnp
from jax import lax
from jax.experimental import pallas as pl
from jax.experimental.pallas import tpu as pltpu
from jax.experimental.pallas import tpu_sc as plsc

_G = 256
_N = 100000
_D = 128
_NC = 2
_NS = 16
_NW = _NC * _NS
_BLK = 128
_NBLKS_TOTAL = (_N + _BLK - 1) // _BLK
_BASE_BLKS = _NBLKS_TOTAL // _NW
_EXTRA = _NBLKS_TOTAL - _BASE_BLKS * _NW
_MAX_BLKS = _BASE_BLKS + 1
_CHUNK = _MAX_BLKS * _BLK
_ZROWS = _G // _NS


def _sc_segment_sum(x, batch):
    mesh = plsc.VectorSubcoreMesh(core_axis_name="c", subcore_axis_name="s")

    @functools.partial(
        pl.kernel,
        out_type=jax.ShapeDtypeStruct((_NC, _G, _D), jnp.float32),
        mesh=mesh,
        scratch_types=[
            pltpu.VMEM((2, _BLK, _D), jnp.float32),
            pltpu.VMEM((2, _BLK), jnp.int32),
            pltpu.VMEM((_ZROWS, _D), jnp.float32),
            pltpu.VMEM_SHARED((_G + 8, _D), jnp.float32),
            pltpu.SemaphoreType.DMA((2,)),
            pltpu.SemaphoreType.DMA((2,)),
        ],
    )
    def seg_sum(x_hbm, b_hbm, out_hbm, xbuf, idx2, zbuf, acc, gsem, isem):
        cid = lax.axis_index("c")
        sid = lax.axis_index("s")
        wid = sid * _NC + cid

        zeros = jnp.zeros((16,), jnp.float32)

        def zrow(j, _):
            for i in range(_D // 16):
                zbuf[j, pl.ds(i * 16, 16)] = zeros
            return 0

        lax.fori_loop(0, _ZROWS, zrow, 0)

        base = _BASE_BLKS * wid + jnp.minimum(wid, _EXTRA)
        nblk = jnp.where(wid < _EXTRA, _MAX_BLKS, _BASE_BLKS)

        def xstart(b):
            return jnp.minimum((base + b) * _BLK, _N - _BLK)

        def start_io(b, slot):
            pltpu.async_copy(x_hbm.at[pl.ds(xstart(b), _BLK)],
                             xbuf.at[slot], gsem.at[slot])
            pltpu.async_copy(b_hbm.at[pl.ds(xstart(b), _BLK)],
                             idx2.at[slot], isem.at[slot])

        def wait_io(b, slot):
            pltpu.make_async_copy(x_hbm.at[pl.ds(xstart(b), _BLK)],
                                  xbuf.at[slot], gsem.at[slot]).wait()
            pltpu.make_async_copy(b_hbm.at[pl.ds(xstart(b), _BLK)],
                                  idx2.at[slot], isem.at[slot]).wait()

        start_io(0, 0)
        pltpu.sync_copy(zbuf, acc.at[pl.ds(sid * _ZROWS, _ZROWS)])
        plsc.subcore_barrier()

        def body(b, _):
            slot = lax.rem(b, 2)

            @pl.when(b < nblk)
            def _process():
                gstart = (base + b) * _BLK
                xs = xstart(b)
                wait_io(b, slot)

                @pl.when(xs != gstart)
                def _fixup():
                    for i in range(_BLK // 16):
                        r = xs + i * 16 + lax.iota(jnp.int32, 16)
                        v = idx2[slot, pl.ds(i * 16, 16)]
                        idx2[slot, pl.ds(i * 16, 16)] = (
                            jnp.where(r >= gstart, v, _G))

                @pl.when(b + 1 < nblk)
                def _prefetch():
                    start_io(b + 1, lax.rem(b + 1, 2))

                pltpu.sync_copy(xbuf.at[slot], acc.at[idx2.at[slot]], add=True)

            return 0

        lax.fori_loop(0, _MAX_BLKS, body, 0)

        plsc.subcore_barrier()

        @pl.when(sid == 0)
        def _readout():
            pltpu.sync_copy(acc.at[pl.ds(0, _G)], out_hbm.at[cid])

    return seg_sum(x, batch)


def _tc_mlp(partials, W1, b1, W2p, b2):
    def mlp(p_ref, w1_ref, b1_ref, w2_ref, b2_ref, o_ref):
        emb = p_ref[0] + p_ref[1]
        h = jnp.maximum(
            jnp.dot(emb, w1_ref[...], preferred_element_type=jnp.float32)
            + b1_ref[...], 0.0)
        o_ref[...] = (
            jnp.dot(h, w2_ref[...], preferred_element_type=jnp.float32)
            + b2_ref[...])

    return pl.pallas_call(
        mlp,
        out_shape=jax.ShapeDtypeStruct((_G, _D), jnp.float32),
    )(partials, W1, b1, W2p, b2)


def kernel(x, batch, y, W1, b1, W2, b2):
    partials = _sc_segment_sum(x, batch.astype(jnp.int32))
    W2p = jnp.pad(W2, ((0, 0), (0, _D - W2.shape[1])))
    b2p = jnp.pad(b2, (0, _D - b2.shape[0]))
    out = _tc_mlp(partials, W1, b1.reshape(1, _D), W2p, b2p.reshape(1, _D))
    pred = out[:, : W2.shape[1]]
    return (pred, y)


# speedup vs baseline: 1.0210x; 1.0210x over previous
import functools

import jax
import jax.numpy as jnp
from jax import lax
from jax.experimental import pallas as pl
from jax.experimental.pallas import tpu as pltpu
from jax.experimental.pallas import tpu_sc as plsc

_G = 256
_N = 100000
_D = 128
_NC = 2
_NS = 16
_NW = _NC * _NS
_BLK = 128
_NBLKS_TOTAL = (_N + _BLK - 1) // _BLK
_BASE_BLKS = _NBLKS_TOTAL // _NW
_EXTRA = _NBLKS_TOTAL - _BASE_BLKS * _NW
_MAX_BLKS = _BASE_BLKS + 1
_CHUNK = _MAX_BLKS * _BLK
_ZROWS = _G // _NS


def _sc_segment_sum(x, batch):
    mesh = plsc.VectorSubcoreMesh(core_axis_name="c", subcore_axis_name="s")

    @functools.partial(
        pl.kernel,
        out_type=jax.ShapeDtypeStruct((_NC, _G, _D), jnp.float32),
        mesh=mesh,
        scratch_types=[
            pltpu.VMEM((2, _BLK, _D), jnp.float32),
            pltpu.VMEM((2, _BLK), jnp.int32),
            pltpu.VMEM((_ZROWS, _D), jnp.float32),
            pltpu.VMEM((8, _D), jnp.float32),
            pltpu.VMEM((8,), jnp.int32),
            pltpu.VMEM_SHARED((_G + 8, _D), jnp.float32),
            pltpu.SemaphoreType.DMA((2,)),
            pltpu.SemaphoreType.DMA((2,)),
        ],
    )
    def seg_sum(x_hbm, b_hbm, out_hbm, xbuf, idx2, zbuf, rbuf, idx1,
                acc, gsem, isem):
        cid = lax.axis_index("c")
        sid = lax.axis_index("s")
        wid = sid * _NC + cid

        zeros = jnp.zeros((16,), jnp.float32)

        def zrow(j, _):
            for i in range(_D // 16):
                zbuf[j, pl.ds(i * 16, 16)] = zeros
            return 0

        lax.fori_loop(0, _ZROWS, zrow, 0)

        def zrow_r(j, _):
            for i in range(_D // 16):
                rbuf[j, pl.ds(i * 16, 16)] = zeros
            return 0

        lax.fori_loop(0, 8, zrow_r, 0)

        base = _BASE_BLKS * wid + jnp.minimum(wid, _EXTRA)
        nblk = jnp.where(wid < _EXTRA, _MAX_BLKS, _BASE_BLKS)

        def xstart(b):
            return jnp.minimum((base + b) * _BLK, _N - _BLK)

        def start_io(b, slot):
            pltpu.async_copy(x_hbm.at[pl.ds(xstart(b), _BLK)],
                             xbuf.at[slot], gsem.at[slot])
            pltpu.async_copy(b_hbm.at[pl.ds(xstart(b), _BLK)],
                             idx2.at[slot], isem.at[slot])

        def wait_io(b, slot):
            pltpu.make_async_copy(x_hbm.at[pl.ds(xstart(b), _BLK)],
                                  xbuf.at[slot], gsem.at[slot]).wait()
            pltpu.make_async_copy(b_hbm.at[pl.ds(xstart(b), _BLK)],
                                  idx2.at[slot], isem.at[slot]).wait()

        start_io(0, 0)
        pltpu.sync_copy(zbuf, acc.at[pl.ds(sid * _ZROWS, _ZROWS)])
        plsc.subcore_barrier()

        def body(b, _):
            slot = lax.rem(b, 2)

            @pl.when(b < nblk)
            def _process():
                gstart = (base + b) * _BLK
                xs = xstart(b)
                wait_io(b, slot)

                @pl.when(xs != gstart)
                def _fixup():
                    for i in range(_BLK // 16):
                        r = xs + i * 16 + lax.iota(jnp.int32, 16)
                        v = idx2[slot, pl.ds(i * 16, 16)]
                        idx2[slot, pl.ds(i * 16, 16)] = (
                            jnp.where(r >= gstart, v, _G))

                @pl.when(b + 1 < nblk)
                def _prefetch():
                    start_io(b + 1, lax.rem(b + 1, 2))

                uniform = (idx2[slot, pl.ds(0, 16)][0]
                           == idx2[slot, pl.ds(_BLK - 16, 16)][15])

                @pl.when(uniform)
                def _reduce_one_segment():
                    def rbody(j, accs):
                        out = accs
                        for u in range(4):
                            out = tuple(
                                a + xbuf[slot, j * 4 + u, pl.ds(i * 16, 16)]
                                for i, a in enumerate(out))
                        return out

                    accs = lax.fori_loop(
                        0, _BLK // 4, rbody,
                        tuple(jnp.zeros((16,), jnp.float32)
                              for _ in range(_D // 16)))
                    for i, a in enumerate(accs):
                        rbuf[0, pl.ds(i * 16, 16)] = a
                    pltpu.sync_copy(b_hbm.at[pl.ds(xs, 8)], idx1)
                    pltpu.sync_copy(rbuf, acc.at[idx1], add=True)

                @pl.when(jnp.logical_not(uniform))
                def _scatter_all():
                    pltpu.sync_copy(xbuf.at[slot], acc.at[idx2.at[slot]],
                                    add=True)

            return 0

        lax.fori_loop(0, _MAX_BLKS, body, 0)

        plsc.subcore_barrier()

        @pl.when(sid == 0)
        def _readout():
            pltpu.sync_copy(acc.at[pl.ds(0, _G)], out_hbm.at[cid])

    return seg_sum(x, batch)


def _tc_mlp(partials, W1, b1, W2p, b2):
    def mlp(p_ref, w1_ref, b1_ref, w2_ref, b2_ref, o_ref):
        emb = p_ref[0] + p_ref[1]
        h = jnp.maximum(
            jnp.dot(emb, w1_ref[...], preferred_element_type=jnp.float32)
            + b1_ref[...], 0.0)
        o_ref[...] = (
            jnp.dot(h, w2_ref[...], preferred_element_type=jnp.float32)
            + b2_ref[...])

    return pl.pallas_call(
        mlp,
        out_shape=jax.ShapeDtypeStruct((_G, _D), jnp.float32),
    )(partials, W1, b1, W2p, b2)


def kernel(x, batch, y, W1, b1, W2, b2):
    partials = _sc_segment_sum(x, batch.astype(jnp.int32))
    W2p = jnp.pad(W2, ((0, 0), (0, _D - W2.shape[1])))
    b2p = jnp.pad(b2, (0, _D - b2.shape[0]))
    out = _tc_mlp(partials, W1, b1.reshape(1, _D), W2p, b2p.reshape(1, _D))
    pred = out[:, : W2.shape[1]]
    return (pred, y)
